# baseline (speedup 1.0000x reference)
import functools

import jax
import jax.numpy as jnp
from jax import lax
from jax.experimental import pallas as pl
from jax.experimental.pallas import tpu as pltpu
from jax.experimental.pallas import tpu_sc as plsc

NC = 2
NS = 16
NW = NC * NS
C = 128
G = 64


def _sc_mesh():
    return plsc.VectorSubcoreMesh(core_axis_name="c", subcore_axis_name="s",
                                  num_cores=NC, num_subcores=NS)


@functools.lru_cache(maxsize=None)
def _make_deg(NP, K):
    RPT = NP // NS

    def body(dst_hbm, w_hbm, z_hbm, out_hbm, dstv, wv, deg_sh):
        cid = lax.axis_index("c")
        sid = lax.axis_index("s")
        wid = cid * NS + sid
        pltpu.sync_copy(z_hbm.at[pl.ds(sid * RPT, RPT)],
                        deg_sh.at[pl.ds(sid * RPT, RPT)])
        pltpu.sync_copy(dst_hbm.at[wid], dstv)
        pltpu.sync_copy(w_hbm.at[wid], wv)
        plsc.subcore_barrier()

        def chunk(j, carry):
            pltpu.sync_copy(wv.at[j], deg_sh.at[dstv.at[j]], add=True)
            return carry

        lax.fori_loop(0, K, chunk, 0)
        plsc.subcore_barrier()
        pltpu.sync_copy(deg_sh.at[pl.ds(sid * RPT, RPT)],
                        out_hbm.at[cid, pl.ds(sid * RPT, RPT)])

    return pl.kernel(
        body,
        out_type=jax.ShapeDtypeStruct((NC, NP), jnp.float32),
        mesh=_sc_mesh(),
        scratch_types=[
            pltpu.VMEM((K, C), jnp.int32),
            pltpu.VMEM((K, C), jnp.float32),
            pltpu.VMEM_SHARED((NP,), jnp.float32),
        ],
    )


@functools.lru_cache(maxsize=None)
def _make_conv(NP, K, HF):
    RPT = NP // NS
    assert K % 2 == 0

    def body(hp_hbm, src_hbm, dst_hbm, w_hbm, z_hbm, out_hbm,
             srcv, dstv, wv, rows0, rows1, srow0, srow1, acc_sh,
             gsem0, gsem1, ssem0, ssem1):
        cid = lax.axis_index("c")
        sid = lax.axis_index("s")
        pltpu.sync_copy(z_hbm.at[pl.ds(sid * RPT, RPT)],
                        acc_sh.at[pl.ds(sid * RPT, RPT)])
        pltpu.sync_copy(src_hbm.at[sid], srcv)
        pltpu.sync_copy(dst_hbm.at[sid], dstv)
        pltpu.sync_copy(w_hbm.at[sid], wv)

        off = cid * NP

        def add_off(q, carry):
            jj = q // (C // 16)
            ee = (q % (C // 16)) * 16
            sl = pl.ds(ee, 16)
            srcv[jj, sl] = srcv[jj, sl] + off
            return carry

        lax.fori_loop(0, K * (C // 16), add_off, 0, unroll=4)
        plsc.subcore_barrier()

        pltpu.async_copy(hp_hbm.at[srcv.at[0]], rows0, gsem0)
        pltpu.async_copy(hp_hbm.at[srcv.at[1]], rows1, gsem1)
        bufs = ((rows0, srow0, gsem0, ssem0), (rows1, srow1, gsem1, ssem1))

        def outer(t, carry):
            for b in range(2):
                rows, srow, gsem, ssem = bufs[b]
                j = 2 * t + b
                pltpu.make_async_copy(hp_hbm.at[pl.ds(0, C)], rows, gsem).wait()

                @pl.when(j >= 2)
                def _drain():
                    pltpu.make_async_copy(srow, acc_sh.at[pl.ds(0, C)],
                                          ssem).wait()

                def egroup(eg, c2):
                    wvec = wv[j, pl.ds(eg * 16, 16)]
                    base = eg * 16
                    for l in range(16):
                        sc = wvec[l]
                        for f in range(HF // 32):
                            a = rows[base + l, pl.ds(f * 32, 16)] * sc
                            bb = rows[base + l, pl.ds(f * 32 + 16, 16)] * sc
                            srow[base + l, pl.ds(f * 32, 32)] = plsc.pack(
                                a, bb, format=plsc.PackFormat.INTERLEAVED)
                    return c2

                lax.fori_loop(0, C // 16, egroup, 0, unroll=2)

                @pl.when(j + 2 < K)
                def _prefetch():
                    pltpu.async_copy(hp_hbm.at[srcv.at[j + 2]], rows, gsem)

                pltpu.async_copy(srow, acc_sh.at[dstv.at[j]], ssem, add=True)
            return carry

        lax.fori_loop(0, K // 2, outer, 0)
        pltpu.make_async_copy(srow0, acc_sh.at[pl.ds(0, C)], ssem0).wait()
        pltpu.make_async_copy(srow1, acc_sh.at[pl.ds(0, C)], ssem1).wait()
        plsc.subcore_barrier()
        pltpu.sync_copy(acc_sh.at[pl.ds(sid * RPT, RPT)],
                        out_hbm.at[cid, pl.ds(sid * RPT, RPT)])

    return pl.kernel(
        body,
        out_type=jax.ShapeDtypeStruct((NC, NP, HF), jnp.bfloat16),
        mesh=_sc_mesh(),
        compiler_params=pltpu.CompilerParams(use_tc_tiling_on_sc=False,
                                             needs_layout_passes=False),
        scratch_types=[
            pltpu.VMEM((K, C), jnp.int32),
            pltpu.VMEM((K, C), jnp.int32),
            pltpu.VMEM((K, C), jnp.float32),
            pltpu.VMEM((C, HF), jnp.float32),
            pltpu.VMEM((C, HF), jnp.float32),
            pltpu.VMEM((C, HF), jnp.bfloat16),
            pltpu.VMEM((C, HF), jnp.bfloat16),
            pltpu.VMEM_SHARED((NP, HF), jnp.bfloat16),
            pltpu.SemaphoreType.DMA,
            pltpu.SemaphoreType.DMA,
            pltpu.SemaphoreType.DMA,
            pltpu.SemaphoreType.DMA,
        ],
    )


def _tc1(degT, xp, W1):
    NP, D = xp.shape
    Hh = W1.shape[0]
    HF = Hh // 2

    def body(deg_ref, x_ref, w1_ref, dis_ref, hp_ref):
        deg = deg_ref[:, 0:1] + deg_ref[:, 1:2] + 1.0
        dis = lax.rsqrt(deg)
        dis_ref[...] = dis
        h = lax.dot_general(x_ref[...], w1_ref[...], (((1,), (1,)), ((), ())),
                            preferred_element_type=jnp.float32) * dis
        hp_ref[0] = h[:, :HF]
        hp_ref[1] = h[:, HF:]

    return pl.pallas_call(
        body,
        out_shape=[jax.ShapeDtypeStruct((NP, 1), jnp.float32),
                   jax.ShapeDtypeStruct((2, NP, HF), jnp.float32)],
    )(degT, xp, W1)


def _tc_mid(acc, hp, dis, b, W2):
    _, NP, HF = hp.shape
    Hh = 2 * HF

    def body(acc_ref, hp_ref, dis_ref, b_ref, w2_ref, pp_ref):
        a = jnp.concatenate(
            [acc_ref[0].astype(jnp.float32) + hp_ref[0],
             acc_ref[1].astype(jnp.float32) + hp_ref[1]],
            axis=1) * dis_ref[...]
        h1 = jnp.maximum(a + b_ref[...][None, :], 0.0)
        p = lax.dot_general(h1, w2_ref[...], (((1,), (1,)), ((), ())),
                            preferred_element_type=jnp.float32) * dis_ref[...]
        pp_ref[0] = p[:, :HF]
        pp_ref[1] = p[:, HF:]

    return pl.pallas_call(
        body,
        out_shape=jax.ShapeDtypeStruct((2, NP, HF), jnp.float32),
    )(acc, hp, dis, b, W2)


def _tc_head(acc, pp, dis, b2, batchp, Wk1, bk1, Wk2, bk2, Wo, bo):
    _, NP, HF = pp.shape
    O = Wo.shape[0]

    def body(acc_ref, pp_ref, dis_ref, b2_ref, batch_ref,
             wk1_ref, bk1_ref, wk2_ref, bk2_ref, wo_ref, bo_ref, out_ref):
        out2 = (jnp.concatenate(
            [acc_ref[0].astype(jnp.float32) + pp_ref[0],
             acc_ref[1].astype(jnp.float32) + pp_ref[1]],
            axis=1) * dis_ref[...] + b2_ref[...][None, :])
        oh = (batch_ref[...][None, :]
              == lax.broadcasted_iota(jnp.int32, (G, NP), 0)).astype(jnp.float32)
        s = lax.dot_general(oh, out2, (((1,), (0,)), ((), ())),
                            preferred_element_type=jnp.float32)
        cnt = jnp.sum(oh, axis=1)[:, None]
        g = s / jnp.maximum(cnt, 1.0)
        h1h = jnp.maximum(
            lax.dot_general(g, wk1_ref[...], (((1,), (1,)), ((), ())),
                            preferred_element_type=jnp.float32)
            + bk1_ref[...][None, :], 0.0)
        h2h = jnp.maximum(
            lax.dot_general(g, wk2_ref[...], (((1,), (1,)), ((), ())),
                            preferred_element_type=jnp.float32)
            + bk2_ref[...][None, :], 0.0)
        cat = jnp.concatenate([h1h, h2h], axis=1)
        out_ref[...] = (lax.dot_general(cat, wo_ref[...], (((1,), (1,)), ((), ())),
                                        preferred_element_type=jnp.float32)
                        + bo_ref[...][None, :])

    return pl.pallas_call(
        body,
        out_shape=jax.ShapeDtypeStruct((G, O), jnp.float32),
    )(acc, pp, dis, b2, batchp, Wk1, bk1, Wk2, bk2, Wo, bo)


def kernel(x, edge_index, edge_weight, batch,
           W1, b1, W2, b2, Wk1, bk1, Wk2, bk2, Wo, bo):
    N, D = x.shape
    Hh = W1.shape[0]
    E = edge_weight.shape[0]

    HF = Hh // 2
    NP = -(-N // (128 * NS)) * (128 * NS)

    EPWd = -(-(-(-E // NW)) // C) * C
    EPd = EPWd * NW
    Kd = EPWd // C
    EPWc = -(-(-(-E // NS)) // (2 * C)) * (2 * C)
    EPc = EPWc * NS
    Kc = EPWc // C

    src = edge_index[0]
    dst = edge_index[1]
    zi = jnp.zeros((max(EPd, EPc) - E,), jnp.int32)
    zf = jnp.zeros((max(EPd, EPc) - E,), jnp.float32)
    dstd = jnp.concatenate([dst, zi[:EPd - E]]).reshape(NW, Kd, C)
    wd = jnp.concatenate([edge_weight, zf[:EPd - E]]).reshape(NW, Kd, C)
    srcc = jnp.concatenate([src, zi[:EPc - E]]).reshape(NS, Kc, C)
    dstc = jnp.concatenate([dst, zi[:EPc - E]]).reshape(NS, Kc, C)
    wc = jnp.concatenate([edge_weight, zf[:EPc - E]]).reshape(NS, Kc, C)
    xp = jnp.concatenate([x, jnp.zeros((NP - N, D), jnp.float32)])
    batchp = jnp.concatenate([batch, jnp.full((NP - N,), G, jnp.int32)])
    zeros1 = jnp.zeros((NP,), jnp.float32)
    zeros2 = jnp.zeros((NP, HF), jnp.bfloat16)

    pidx = []
    for blk in range(HF // 32):
        pidx += [blk * 32 + 2 * i for i in range(16)]
        pidx += [blk * 32 + 2 * i + 1 for i in range(16)]
    perm = jnp.array(pidx, jnp.int32)

    deg_parts = _make_deg(NP, Kd)(dstd, wd, zeros1)
    degT = deg_parts.transpose((1, 0))
    dis, hp = _tc1(degT, xp, W1)
    conv = _make_conv(NP, Kc, HF)
    acc1 = conv(hp.reshape(2 * NP, HF)[:, perm], srcc, dstc, wc, zeros2)
    pp = _tc_mid(acc1, hp, dis, b1, W2)
    acc2 = conv(pp.reshape(2 * NP, HF)[:, perm], srcc, dstc, wc, zeros2)
    return _tc_head(acc2, pp, dis, b2, batchp, Wk1, bk1, Wk2, bk2, Wo, bo)


# speedup vs baseline: 1.3896x; 1.3896x over previous
import functools

import jax
import jax.numpy as jnp
from jax import lax
from jax.experimental import pallas as pl
from jax.experimental.pallas import tpu as pltpu
from jax.experimental.pallas import tpu_sc as plsc

NC = 2
NS = 16
NW = NC * NS
C = 128
G = 64


def _sc_mesh():
    return plsc.VectorSubcoreMesh(core_axis_name="c", subcore_axis_name="s",
                                  num_cores=NC, num_subcores=NS)


@functools.lru_cache(maxsize=None)
def _make_deg(NP, K):
    RPT = NP // NS

    def body(dst_hbm, w_hbm, z_hbm, out_hbm, dstv, wv, deg_sh):
        cid = lax.axis_index("c")
        sid = lax.axis_index("s")
        wid = cid * NS + sid
        pltpu.sync_copy(z_hbm.at[pl.ds(sid * RPT, RPT)],
                        deg_sh.at[pl.ds(sid * RPT, RPT)])
        pltpu.sync_copy(dst_hbm.at[wid], dstv)
        pltpu.sync_copy(w_hbm.at[wid], wv)
        plsc.subcore_barrier()

        def chunk(j, carry):
            pltpu.sync_copy(wv.at[j], deg_sh.at[dstv.at[j]], add=True)
            return carry

        lax.fori_loop(0, K, chunk, 0)
        plsc.subcore_barrier()
        pltpu.sync_copy(deg_sh.at[pl.ds(sid * RPT, RPT)],
                        out_hbm.at[cid, pl.ds(sid * RPT, RPT)])

    return pl.kernel(
        body,
        out_type=jax.ShapeDtypeStruct((NC, NP), jnp.float32),
        mesh=_sc_mesh(),
        scratch_types=[
            pltpu.VMEM((K, C), jnp.int32),
            pltpu.VMEM((K, C), jnp.float32),
            pltpu.VMEM_SHARED((NP,), jnp.float32),
        ],
    )


@functools.lru_cache(maxsize=None)
def _make_conv(NP, K, HF):
    RPT = NP // NS
    assert K % 2 == 0

    def body(hp_hbm, src_hbm, dst_hbm, w_hbm, z_hbm, out_hbm,
             srcv, dstv, wring, rows0, rows1, srow0, srow1, acc_sh,
             gsem0, gsem1, ssem0, ssem1):
        cid = lax.axis_index("c")
        sid = lax.axis_index("s")
        pltpu.sync_copy(z_hbm.at[pl.ds(sid * RPT, RPT)],
                        acc_sh.at[pl.ds(sid * RPT, RPT)])
        pltpu.sync_copy(src_hbm.at[sid], srcv)
        pltpu.sync_copy(dst_hbm.at[sid], dstv)

        off = cid * NP

        def add_off(q, carry):
            jj = q // (C // 16)
            ee = (q % (C // 16)) * 16
            sl = pl.ds(ee, 16)
            srcv[jj, sl] = srcv[jj, sl] + off
            return carry

        lax.fori_loop(0, K * (C // 16), add_off, 0, unroll=4)
        plsc.subcore_barrier()

        pltpu.async_copy(hp_hbm.at[srcv.at[0]], rows0, gsem0)
        pltpu.async_copy(w_hbm.at[sid, 0], wring.at[0], gsem0)
        pltpu.async_copy(hp_hbm.at[srcv.at[1]], rows1, gsem1)
        pltpu.async_copy(w_hbm.at[sid, 1], wring.at[1], gsem1)
        bufs = ((rows0, srow0, gsem0, ssem0), (rows1, srow1, gsem1, ssem1))

        def outer(t, carry):
            for b in range(2):
                rows, srow, gsem, ssem = bufs[b]
                j = 2 * t + b
                pltpu.make_async_copy(hp_hbm.at[pl.ds(0, C)], rows, gsem).wait()
                pltpu.make_async_copy(w_hbm.at[0, 0], wring.at[b], gsem).wait()

                @pl.when(j >= 2)
                def _drain():
                    pltpu.make_async_copy(srow, acc_sh.at[pl.ds(0, C)],
                                          ssem).wait()

                def egroup(eg, c2):
                    wvec = wring[b, pl.ds(eg * 16, 16)]
                    base = eg * 16
                    for l in range(16):
                        sc = wvec[l]
                        for f in range(HF // 16):
                            sl = pl.ds(f * 16, 16)
                            srow[base + l, sl] = rows[base + l, sl] * sc
                    return c2

                lax.fori_loop(0, C // 16, egroup, 0, unroll=2)

                @pl.when(j + 2 < K)
                def _prefetch():
                    pltpu.async_copy(hp_hbm.at[srcv.at[j + 2]], rows, gsem)
                    pltpu.async_copy(w_hbm.at[sid, j + 2], wring.at[b], gsem)

                pltpu.async_copy(srow, acc_sh.at[dstv.at[j]], ssem, add=True)
            return carry

        lax.fori_loop(0, K // 2, outer, 0)
        pltpu.make_async_copy(srow0, acc_sh.at[pl.ds(0, C)], ssem0).wait()
        pltpu.make_async_copy(srow1, acc_sh.at[pl.ds(0, C)], ssem1).wait()
        plsc.subcore_barrier()
        pltpu.sync_copy(acc_sh.at[pl.ds(sid * RPT, RPT)],
                        out_hbm.at[cid, pl.ds(sid * RPT, RPT)])

    return pl.kernel(
        body,
        out_type=jax.ShapeDtypeStruct((NC, NP, HF), jnp.float32),
        mesh=_sc_mesh(),
        compiler_params=pltpu.CompilerParams(use_tc_tiling_on_sc=False),
        scratch_types=[
            pltpu.VMEM((K, C), jnp.int32),
            pltpu.VMEM((K, C), jnp.int32),
            pltpu.VMEM((2, C), jnp.float32),
            pltpu.VMEM((C, HF), jnp.float32),
            pltpu.VMEM((C, HF), jnp.float32),
            pltpu.VMEM((C, HF), jnp.float32),
            pltpu.VMEM((C, HF), jnp.float32),
            pltpu.VMEM_SHARED((NP, HF), jnp.float32),
            pltpu.SemaphoreType.DMA,
            pltpu.SemaphoreType.DMA,
            pltpu.SemaphoreType.DMA,
            pltpu.SemaphoreType.DMA,
        ],
    )


def _tc1(degT, xp, W1):
    NP, D = xp.shape
    Hh = W1.shape[0]
    HF = Hh // 2

    def body(deg_ref, x_ref, w1_ref, dis_ref, hp_ref):
        deg = deg_ref[:, 0:1] + deg_ref[:, 1:2] + 1.0
        dis = lax.rsqrt(deg)
        dis_ref[...] = dis
        h = lax.dot_general(x_ref[...], w1_ref[...], (((1,), (1,)), ((), ())),
                            preferred_element_type=jnp.float32) * dis
        hp_ref[0] = h[:, :HF]
        hp_ref[1] = h[:, HF:]

    return pl.pallas_call(
        body,
        out_shape=[jax.ShapeDtypeStruct((NP, 1), jnp.float32),
                   jax.ShapeDtypeStruct((2, NP, HF), jnp.float32)],
    )(degT, xp, W1)


def _tc_mid(acc, hp, dis, b, W2):
    _, NP, HF = hp.shape
    Hh = 2 * HF

    def body(acc_ref, hp_ref, dis_ref, b_ref, w2_ref, pp_ref):
        a = jnp.concatenate([acc_ref[0] + hp_ref[0],
                             acc_ref[1] + hp_ref[1]], axis=1) * dis_ref[...]
        h1 = jnp.maximum(a + b_ref[...][None, :], 0.0)
        p = lax.dot_general(h1, w2_ref[...], (((1,), (1,)), ((), ())),
                            preferred_element_type=jnp.float32) * dis_ref[...]
        pp_ref[0] = p[:, :HF]
        pp_ref[1] = p[:, HF:]

    return pl.pallas_call(
        body,
        out_shape=jax.ShapeDtypeStruct((2, NP, HF), jnp.float32),
    )(acc, hp, dis, b, W2)


def _tc_head(acc, pp, dis, b2, batchp, Wk1, bk1, Wk2, bk2, Wo, bo):
    _, NP, HF = pp.shape
    O = Wo.shape[0]

    def body(acc_ref, pp_ref, dis_ref, b2_ref, batch_ref,
             wk1_ref, bk1_ref, wk2_ref, bk2_ref, wo_ref, bo_ref, out_ref):
        out2 = (jnp.concatenate([acc_ref[0] + pp_ref[0],
                                 acc_ref[1] + pp_ref[1]], axis=1)
                * dis_ref[...] + b2_ref[...][None, :])
        oh = (batch_ref[...][None, :]
              == lax.broadcasted_iota(jnp.int32, (G, NP), 0)).astype(jnp.float32)
        s = lax.dot_general(oh, out2, (((1,), (0,)), ((), ())),
                            preferred_element_type=jnp.float32)
        cnt = jnp.sum(oh, axis=1)[:, None]
        g = s / jnp.maximum(cnt, 1.0)
        h1h = jnp.maximum(
            lax.dot_general(g, wk1_ref[...], (((1,), (1,)), ((), ())),
                            preferred_element_type=jnp.float32)
            + bk1_ref[...][None, :], 0.0)
        h2h = jnp.maximum(
            lax.dot_general(g, wk2_ref[...], (((1,), (1,)), ((), ())),
                            preferred_element_type=jnp.float32)
            + bk2_ref[...][None, :], 0.0)
        cat = jnp.concatenate([h1h, h2h], axis=1)
        out_ref[...] = (lax.dot_general(cat, wo_ref[...], (((1,), (1,)), ((), ())),
                                        preferred_element_type=jnp.float32)
                        + bo_ref[...][None, :])

    return pl.pallas_call(
        body,
        out_shape=jax.ShapeDtypeStruct((G, O), jnp.float32),
    )(acc, pp, dis, b2, batchp, Wk1, bk1, Wk2, bk2, Wo, bo)


def kernel(x, edge_index, edge_weight, batch,
           W1, b1, W2, b2, Wk1, bk1, Wk2, bk2, Wo, bo):
    N, D = x.shape
    Hh = W1.shape[0]
    E = edge_weight.shape[0]

    HF = Hh // 2
    NP = -(-N // (128 * NS)) * (128 * NS)

    EPWd = -(-(-(-E // NW)) // C) * C
    EPd = EPWd * NW
    Kd = EPWd // C
    EPWc = -(-(-(-E // NS)) // (2 * C)) * (2 * C)
    EPc = EPWc * NS
    Kc = EPWc // C

    src = edge_index[0]
    dst = edge_index[1]
    zi = jnp.zeros((max(EPd, EPc) - E,), jnp.int32)
    zf = jnp.zeros((max(EPd, EPc) - E,), jnp.float32)
    dstd = jnp.concatenate([dst, zi[:EPd - E]]).reshape(NW, Kd, C)
    wd = jnp.concatenate([edge_weight, zf[:EPd - E]]).reshape(NW, Kd, C)
    srcc = jnp.concatenate([src, zi[:EPc - E]]).reshape(NS, Kc, C)
    dstc = jnp.concatenate([dst, zi[:EPc - E]]).reshape(NS, Kc, C)
    wc = jnp.concatenate([edge_weight, zf[:EPc - E]]).reshape(NS, Kc, C)
    xp = jnp.concatenate([x, jnp.zeros((NP - N, D), jnp.float32)])
    batchp = jnp.concatenate([batch, jnp.full((NP - N,), G, jnp.int32)])
    zeros1 = jnp.zeros((NP,), jnp.float32)
    zeros2 = jnp.zeros((NP, HF), jnp.float32)

    deg_parts = _make_deg(NP, Kd)(dstd, wd, zeros1)
    degT = deg_parts.transpose((1, 0))
    dis, hp = _tc1(degT, xp, W1)
    conv = _make_conv(NP, Kc, HF)
    acc1 = conv(hp.reshape(2 * NP, HF), srcc, dstc, wc, zeros2)
    pp = _tc_mid(acc1, hp, dis, b1, W2)
    acc2 = conv(pp.reshape(2 * NP, HF), srcc, dstc, wc, zeros2)
    return _tc_head(acc2, pp, dis, b2, batchp, Wk1, bk1, Wk2, bk2, Wo, bo)
